# baseline (speedup 1.0000x reference)
import numpy as np
import jax
import jax.numpy as jnp
from jax.experimental import pallas as pl
from jax.experimental.pallas import tpu as pltpu

_B, _NODE, _C = 4, 32, 256
_N = _NODE * _NODE
_K1 = 11
_KS, _STRIDE = 5, 2
_EPS = 1e-5
_BR = 128
_NB = _N // _BR


def _local_parts():
    size, ks, stride = _NODE, _KS, _STRIDE
    inp = np.arange(size * size).reshape(size, size)
    patches = []
    for i in range(0, size - ks + 1, stride):
        for j in range(0, size - ks + 1, stride):
            patches.append(inp[i:i + ks, j:j + ks].reshape(-1))
    inp_unf = np.stack(patches, axis=0)
    edge = inp_unf.shape[0]
    H = np.zeros((size * size, edge), dtype=np.float32)
    for e in range(edge):
        H[inp_unf[e], e] = 1.0
    L = (H @ H.T) / float(ks * ks)
    cloc = H.sum(axis=1)
    return L.astype(np.float32), cloc.astype(np.float32)


_LOC_L, _LOC_CLOC = _local_parts()


def _members_from_threshold(st, v, kvec):
    n, p = st.shape
    lt = st < v
    c_lt = jnp.sum(lt.astype(jnp.int32), axis=0, keepdims=True)
    eqm = (st == v).astype(jnp.int32)
    c_eq = jnp.sum(eqm, axis=0, keepdims=True)
    simple = jnp.all(c_lt + c_eq == kvec)

    def _zeros():
        return jnp.zeros((n, p), jnp.int32)

    def _scan():
        csum = eqm
        d = 1
        while d < n:
            shifted = jnp.concatenate(
                [jnp.zeros((d, p), jnp.int32), csum[: n - d, :]], axis=0)
            csum = csum + shifted
            d *= 2
        return csum - eqm

    excl = jax.lax.cond(simple, _zeros, _scan)
    tie_take = eqm.astype(jnp.bool_) & (excl < (kvec - c_lt))
    return lt | tie_take


def _threshold_extract(st, k):
    rem = st
    removed = jnp.zeros((1, st.shape[1]), jnp.int32)
    v = jnp.full((1, st.shape[1]), -2147483648, jnp.int32)
    for _ in range(k):
        cur = jnp.min(rem, axis=0, keepdims=True)
        isv = rem == cur
        ccur = jnp.sum(isv.astype(jnp.int32), axis=0, keepdims=True)
        upd = removed < k
        v = jnp.where(upd, cur, v)
        removed = removed + jnp.where(upd, ccur, 0)
        rem = jnp.where(isv, jnp.int32(2147483647), rem)
    return v


def _threshold_radix(st, kvec):
    neg = (st < 0).astype(jnp.int32)
    c0 = jnp.sum(neg, axis=0, keepdims=True)
    cond = kvec <= c0
    prefix = jnp.where(cond, jnp.int32(-2147483648), jnp.int32(0))
    kk = jnp.where(cond, kvec, kvec - c0)
    for b in range(30, -1, -1):
        eq = (st >> (b + 1)) == (prefix >> (b + 1))
        bit0 = (st & (1 << b)) == 0
        c0 = jnp.sum((eq & bit0).astype(jnp.int32), axis=0, keepdims=True)
        cond = kk <= c0
        prefix = jnp.where(cond, prefix, prefix | (1 << b))
        kk = jnp.where(cond, kk, kk - c0)
    return prefix


def _dist_block(xb, xf):
    xxt = jax.lax.dot_general(xf, xb, (((1,), (1,)), ((), ())),
                              preferred_element_type=jnp.float32)
    xsqp = jax.lax.dot_general(jnp.ones((1, _C), jnp.float32), xb * xb,
                               (((1,), (1,)), ((), ())),
                               preferred_element_type=jnp.float32)
    xsqn = jnp.sum(xf * xf, axis=1, keepdims=True)
    D = xsqp + (-2.0 * xxt) + xsqn
    bits = jax.lax.bitcast_convert_type(D, jnp.int32)
    return jnp.where(bits < 0, bits ^ jnp.int32(0x7FFFFFFF), bits)


def _dist_deg_body(xb_ref, xf_ref, dv_ref):
    st = _dist_block(xb_ref[0], xf_ref[0])
    kvec = jnp.full((1, _BR), _K1, jnp.int32)
    v = _threshold_extract(st, _K1)
    m1 = _members_from_threshold(st, v, kvec).astype(jnp.float32)
    part = jnp.sum(m1, axis=1, keepdims=True)

    @pl.when(pl.program_id(1) == 0)
    def _init():
        dv_ref[0] = part

    @pl.when(pl.program_id(1) != 0)
    def _acc():
        dv_ref[0] += part


def _inc_gp_body(xb_ref, dvr_ref, x_ref, Wc_ref, bc_ref, L_ref, clc_ref,
                 clr_ref, P_ref, S_s, h_s, dvc_s, dvr_s):
    j = pl.program_id(1)

    @pl.when(j == 0)
    def _h():
        h = jax.lax.dot_general(x_ref[0], Wc_ref[...],
                                (((1,), (1,)), ((), ())),
                                preferred_element_type=jnp.float32)
        h_s[...] = h + bc_ref[...]

    kvec = dvr_ref[0].astype(jnp.int32)
    st = _dist_block(xb_ref[0], x_ref[0])
    v = _threshold_radix(st, kvec)
    m2 = _members_from_threshold(st, v, kvec).astype(jnp.float32)
    ri = jax.lax.broadcasted_iota(jnp.int32, (_N, _BR), 0)
    ci = jax.lax.broadcasted_iota(jnp.int32, (_N, _BR), 1) + j * _BR
    nh = jnp.where(ri == ci, 1.0, m2)
    de = jnp.sum(nh, axis=0, keepdims=True)
    aw = nh * (1.0 / de)
    spart = jax.lax.dot_general(aw, nh, (((1,), (1,)), ((), ())),
                                preferred_element_type=jnp.float32)
    dcol = jnp.sum(nh, axis=1, keepdims=True)
    drow = jax.lax.dot_general(jnp.ones((1, _BR), jnp.float32), nh,
                               (((1,), (1,)), ((), ())),
                               preferred_element_type=jnp.float32)

    @pl.when(j == 0)
    def _init():
        S_s[...] = spart
        dvc_s[...] = dcol
        dvr_s[...] = drow

    @pl.when(j != 0)
    def _acc():
        S_s[...] += spart
        dvc_s[...] += dcol
        dvr_s[...] += drow

    @pl.when(j == _NB - 1)
    def _fin():
        sc = jax.lax.rsqrt(dvc_s[...] + clc_ref[...])
        sr = jax.lax.rsqrt(dvr_s[...] + clr_ref[...])
        G = (S_s[...] + L_ref[...]) * sc * sr
        P_ref[0] = jax.lax.dot_general(G, h_s[...], (((1,), (0,)), ((), ())),
                                       preferred_element_type=jnp.float32)


def _bn_body(P_ref, x_ref, gamma_ref, beta_ref, o_ref):
    P = jnp.reshape(P_ref[...], (_B * _N, _C))
    m = jnp.mean(P, axis=0, keepdims=True)
    d = P - m
    var = jnp.mean(d * d, axis=0, keepdims=True)
    hn = d / jnp.sqrt(var + _EPS) * gamma_ref[...] + beta_ref[...]
    hr = jnp.maximum(hn, 0.0)
    o_ref[...] = jnp.reshape(hr, (_B, _N, _C)) + x_ref[...]


@jax.jit
def kernel(x, Wc, bc, gamma, beta):
    L = jnp.asarray(_LOC_L)
    clc = jnp.asarray(_LOC_CLOC).reshape(_N, 1)
    clr = jnp.asarray(_LOC_CLOC).reshape(1, _N)

    dv_col = pl.pallas_call(
        _dist_deg_body,
        grid=(_B, _NB),
        in_specs=[
            pl.BlockSpec((1, _BR, _C), lambda b, j: (b, j, 0)),
            pl.BlockSpec((1, _N, _C), lambda b, j: (b, 0, 0)),
        ],
        out_specs=pl.BlockSpec((1, _N, 1), lambda b, j: (b, 0, 0)),
        out_shape=jax.ShapeDtypeStruct((_B, _N, 1), jnp.float32),
    )(x, x)
    dv_row = jnp.swapaxes(dv_col, 1, 2)

    P = pl.pallas_call(
        _inc_gp_body,
        grid=(_B, _NB),
        in_specs=[
            pl.BlockSpec((1, _BR, _C), lambda b, j: (b, j, 0)),
            pl.BlockSpec((1, 1, _BR), lambda b, j: (b, 0, j)),
            pl.BlockSpec((1, _N, _C), lambda b, j: (b, 0, 0)),
            pl.BlockSpec((_C, _C), lambda b, j: (0, 0)),
            pl.BlockSpec((1, _C), lambda b, j: (0, 0)),
            pl.BlockSpec((_N, _N), lambda b, j: (0, 0)),
            pl.BlockSpec((_N, 1), lambda b, j: (0, 0)),
            pl.BlockSpec((1, _N), lambda b, j: (0, 0)),
        ],
        out_specs=pl.BlockSpec((1, _N, _C), lambda b, j: (b, 0, 0)),
        out_shape=jax.ShapeDtypeStruct((_B, _N, _C), jnp.float32),
        scratch_shapes=[
            pltpu.VMEM((_N, _N), jnp.float32),
            pltpu.VMEM((_N, _C), jnp.float32),
            pltpu.VMEM((_N, 1), jnp.float32),
            pltpu.VMEM((1, _N), jnp.float32),
        ],
    )(x, dv_row, x, Wc, bc.reshape(1, _C), L, clc, clr)

    out = pl.pallas_call(
        _bn_body,
        out_shape=jax.ShapeDtypeStruct((_B, _N, _C), jnp.float32),
    )(P, x, gamma.reshape(1, _C), beta.reshape(1, _C))
    return out


# speedup vs baseline: 1.0570x; 1.0570x over previous
import numpy as np
import jax
import jax.numpy as jnp
from jax.experimental import pallas as pl
from jax.experimental.pallas import tpu as pltpu

_B, _NODE, _C = 4, 32, 256
_N = _NODE * _NODE
_K1 = 11
_KS, _STRIDE = 5, 2
_EPS = 1e-5
_BR = 256
_NB = _N // _BR


def _local_parts():
    size, ks, stride = _NODE, _KS, _STRIDE
    inp = np.arange(size * size).reshape(size, size)
    patches = []
    for i in range(0, size - ks + 1, stride):
        for j in range(0, size - ks + 1, stride):
            patches.append(inp[i:i + ks, j:j + ks].reshape(-1))
    inp_unf = np.stack(patches, axis=0)
    edge = inp_unf.shape[0]
    H = np.zeros((size * size, edge), dtype=np.float32)
    for e in range(edge):
        H[inp_unf[e], e] = 1.0
    L = (H @ H.T) / float(ks * ks)
    cloc = H.sum(axis=1)
    return L.astype(np.float32), cloc.astype(np.float32)


_LOC_L, _LOC_CLOC = _local_parts()


def _members_from_threshold(st, v, kvec):
    n, p = st.shape
    lt = st < v
    c_lt = jnp.sum(lt.astype(jnp.int32), axis=0, keepdims=True)
    eqm = (st == v).astype(jnp.int32)
    c_eq = jnp.sum(eqm, axis=0, keepdims=True)
    simple = jnp.all(c_lt + c_eq == kvec)

    def _zeros():
        return jnp.zeros((n, p), jnp.int32)

    def _scan():
        csum = eqm
        d = 1
        while d < n:
            shifted = jnp.concatenate(
                [jnp.zeros((d, p), jnp.int32), csum[: n - d, :]], axis=0)
            csum = csum + shifted
            d *= 2
        return csum - eqm

    excl = jax.lax.cond(simple, _zeros, _scan)
    tie_take = eqm.astype(jnp.bool_) & (excl < (kvec - c_lt))
    return lt | tie_take


def _threshold_extract(st, k):
    rem = st
    removed = jnp.zeros((1, st.shape[1]), jnp.int32)
    v = jnp.full((1, st.shape[1]), -2147483648, jnp.int32)
    for _ in range(k):
        cur = jnp.min(rem, axis=0, keepdims=True)
        isv = rem == cur
        ccur = jnp.sum(isv.astype(jnp.int32), axis=0, keepdims=True)
        upd = removed < k
        v = jnp.where(upd, cur, v)
        removed = removed + jnp.where(upd, ccur, 0)
        rem = jnp.where(isv, jnp.int32(2147483647), rem)
    return v


def _threshold_radix(st, kvec):
    neg = (st < 0).astype(jnp.int32)
    c0 = jnp.sum(neg, axis=0, keepdims=True)
    cond = kvec <= c0
    prefix = jnp.where(cond, jnp.int32(-2147483648), jnp.int32(0))
    kk = jnp.where(cond, kvec, kvec - c0)
    for b in range(30, -1, -1):
        eq = (st >> (b + 1)) == (prefix >> (b + 1))
        bit0 = (st & (1 << b)) == 0
        c0 = jnp.sum((eq & bit0).astype(jnp.int32), axis=0, keepdims=True)
        cond = kk <= c0
        prefix = jnp.where(cond, prefix, prefix | (1 << b))
        kk = jnp.where(cond, kk, kk - c0)
    return prefix


def _dist_block(xb, xf):
    xxt = jax.lax.dot_general(xf, xb, (((1,), (1,)), ((), ())),
                              preferred_element_type=jnp.float32)
    xsqp = jax.lax.dot_general(jnp.ones((1, _C), jnp.float32), xb * xb,
                               (((1,), (1,)), ((), ())),
                               preferred_element_type=jnp.float32)
    xsqn = jnp.sum(xf * xf, axis=1, keepdims=True)
    D = xsqp + (-2.0 * xxt) + xsqn
    bits = jax.lax.bitcast_convert_type(D, jnp.int32)
    return jnp.where(bits < 0, bits ^ jnp.int32(0x7FFFFFFF), bits)


def _dist_deg_body(xb_ref, xf_ref, dv_ref):
    st = _dist_block(xb_ref[0], xf_ref[0])
    kvec = jnp.full((1, _BR), _K1, jnp.int32)
    v = _threshold_extract(st, _K1)
    m1 = _members_from_threshold(st, v, kvec).astype(jnp.float32)
    part = jnp.sum(m1, axis=1, keepdims=True)

    @pl.when(pl.program_id(1) == 0)
    def _init():
        dv_ref[0] = part

    @pl.when(pl.program_id(1) != 0)
    def _acc():
        dv_ref[0] += part


def _inc_gp_body(xb_ref, dvr_ref, x_ref, Wc_ref, bc_ref, L_ref, clc_ref,
                 clr_ref, P_ref, S_s, h_s, dvc_s, dvr_s):
    j = pl.program_id(1)

    @pl.when(j == 0)
    def _h():
        h = jax.lax.dot_general(x_ref[0], Wc_ref[...],
                                (((1,), (1,)), ((), ())),
                                preferred_element_type=jnp.float32)
        h_s[...] = h + bc_ref[...]

    kvec = dvr_ref[0].astype(jnp.int32)
    st = _dist_block(xb_ref[0], x_ref[0])
    v = _threshold_radix(st, kvec)
    m2 = _members_from_threshold(st, v, kvec).astype(jnp.float32)
    ri = jax.lax.broadcasted_iota(jnp.int32, (_N, _BR), 0)
    ci = jax.lax.broadcasted_iota(jnp.int32, (_N, _BR), 1) + j * _BR
    nh = jnp.where(ri == ci, 1.0, m2)
    de = jnp.sum(nh, axis=0, keepdims=True)
    aw = nh * (1.0 / de)
    spart = jax.lax.dot_general(aw, nh, (((1,), (1,)), ((), ())),
                                preferred_element_type=jnp.float32)
    dcol = jnp.sum(nh, axis=1, keepdims=True)
    drow = jax.lax.dot_general(jnp.ones((1, _BR), jnp.float32), nh,
                               (((1,), (1,)), ((), ())),
                               preferred_element_type=jnp.float32)

    @pl.when(j == 0)
    def _init():
        S_s[...] = spart
        dvc_s[...] = dcol
        dvr_s[...] = drow

    @pl.when(j != 0)
    def _acc():
        S_s[...] += spart
        dvc_s[...] += dcol
        dvr_s[...] += drow

    @pl.when(j == _NB - 1)
    def _fin():
        sc = jax.lax.rsqrt(dvc_s[...] + clc_ref[...])
        sr = jax.lax.rsqrt(dvr_s[...] + clr_ref[...])
        G = (S_s[...] + L_ref[...]) * sc * sr
        P_ref[0] = jax.lax.dot_general(G, h_s[...], (((1,), (0,)), ((), ())),
                                       preferred_element_type=jnp.float32)


def _bn_body(P_ref, x_ref, gamma_ref, beta_ref, o_ref):
    P = jnp.reshape(P_ref[...], (_B * _N, _C))
    m = jnp.mean(P, axis=0, keepdims=True)
    d = P - m
    var = jnp.mean(d * d, axis=0, keepdims=True)
    hn = d / jnp.sqrt(var + _EPS) * gamma_ref[...] + beta_ref[...]
    hr = jnp.maximum(hn, 0.0)
    o_ref[...] = jnp.reshape(hr, (_B, _N, _C)) + x_ref[...]


@jax.jit
def kernel(x, Wc, bc, gamma, beta):
    L = jnp.asarray(_LOC_L)
    clc = jnp.asarray(_LOC_CLOC).reshape(_N, 1)
    clr = jnp.asarray(_LOC_CLOC).reshape(1, _N)

    dv_col = pl.pallas_call(
        _dist_deg_body,
        grid=(_B, _NB),
        in_specs=[
            pl.BlockSpec((1, _BR, _C), lambda b, j: (b, j, 0)),
            pl.BlockSpec((1, _N, _C), lambda b, j: (b, 0, 0)),
        ],
        out_specs=pl.BlockSpec((1, _N, 1), lambda b, j: (b, 0, 0)),
        out_shape=jax.ShapeDtypeStruct((_B, _N, 1), jnp.float32),
    )(x, x)
    dv_row = jnp.swapaxes(dv_col, 1, 2)

    P = pl.pallas_call(
        _inc_gp_body,
        grid=(_B, _NB),
        in_specs=[
            pl.BlockSpec((1, _BR, _C), lambda b, j: (b, j, 0)),
            pl.BlockSpec((1, 1, _BR), lambda b, j: (b, 0, j)),
            pl.BlockSpec((1, _N, _C), lambda b, j: (b, 0, 0)),
            pl.BlockSpec((_C, _C), lambda b, j: (0, 0)),
            pl.BlockSpec((1, _C), lambda b, j: (0, 0)),
            pl.BlockSpec((_N, _N), lambda b, j: (0, 0)),
            pl.BlockSpec((_N, 1), lambda b, j: (0, 0)),
            pl.BlockSpec((1, _N), lambda b, j: (0, 0)),
        ],
        out_specs=pl.BlockSpec((1, _N, _C), lambda b, j: (b, 0, 0)),
        out_shape=jax.ShapeDtypeStruct((_B, _N, _C), jnp.float32),
        scratch_shapes=[
            pltpu.VMEM((_N, _N), jnp.float32),
            pltpu.VMEM((_N, _C), jnp.float32),
            pltpu.VMEM((_N, 1), jnp.float32),
            pltpu.VMEM((1, _N), jnp.float32),
        ],
    )(x, dv_row, x, Wc, bc.reshape(1, _C), L, clc, clr)

    out = pl.pallas_call(
        _bn_body,
        out_shape=jax.ShapeDtypeStruct((_B, _N, _C), jnp.float32),
    )(P, x, gamma.reshape(1, _C), beta.reshape(1, _C))
    return out


# speedup vs baseline: 1.5499x; 1.4663x over previous
import numpy as np
import jax
import jax.numpy as jnp
from jax.experimental import pallas as pl
from jax.experimental.pallas import tpu as pltpu

_B, _NODE, _C = 4, 32, 256
_N = _NODE * _NODE
_K1 = 11
_KS, _STRIDE = 5, 2
_EPS = 1e-5
_BR = 256
_NB = _N // _BR


def _local_parts():
    size, ks, stride = _NODE, _KS, _STRIDE
    inp = np.arange(size * size).reshape(size, size)
    patches = []
    for i in range(0, size - ks + 1, stride):
        for j in range(0, size - ks + 1, stride):
            patches.append(inp[i:i + ks, j:j + ks].reshape(-1))
    inp_unf = np.stack(patches, axis=0)
    edge = inp_unf.shape[0]
    H = np.zeros((size * size, edge), dtype=np.float32)
    for e in range(edge):
        H[inp_unf[e], e] = 1.0
    L = (H @ H.T) / float(ks * ks)
    cloc = H.sum(axis=1)
    return L.astype(np.float32), cloc.astype(np.float32)


_LOC_L, _LOC_CLOC = _local_parts()


def _members_from_threshold(st, v, kvec):
    n, p = st.shape
    lt = st < v
    c_lt = jnp.sum(lt.astype(jnp.int32), axis=0, keepdims=True)
    eqm = (st == v).astype(jnp.int32)
    c_eq = jnp.sum(eqm, axis=0, keepdims=True)
    simple = jnp.all(c_lt + c_eq == kvec)

    def _zeros():
        return jnp.zeros((n, p), jnp.int32)

    def _scan():
        csum = eqm
        d = 1
        while d < n:
            shifted = jnp.concatenate(
                [jnp.zeros((d, p), jnp.int32), csum[: n - d, :]], axis=0)
            csum = csum + shifted
            d *= 2
        return csum - eqm

    excl = jax.lax.cond(simple, _zeros, _scan)
    tie_take = eqm.astype(jnp.bool_) & (excl < (kvec - c_lt))
    return lt | tie_take


def _threshold_extract(st, k):
    rem = st
    removed = jnp.zeros((1, st.shape[1]), jnp.int32)
    v = jnp.full((1, st.shape[1]), -2147483648, jnp.int32)
    for _ in range(k):
        cur = jnp.min(rem, axis=0, keepdims=True)
        isv = rem == cur
        ccur = jnp.sum(isv.astype(jnp.int32), axis=0, keepdims=True)
        upd = removed < k
        v = jnp.where(upd, cur, v)
        removed = removed + jnp.where(upd, ccur, 0)
        rem = jnp.where(isv, jnp.int32(2147483647), rem)
    return v


def _threshold_radix(st, kvec):
    neg = (st < 0).astype(jnp.int32)
    c0 = jnp.sum(neg, axis=0, keepdims=True)
    cond = kvec <= c0
    prefix = jnp.where(cond, jnp.int32(-2147483648), jnp.int32(0))
    kk = jnp.where(cond, kvec, kvec - c0)
    for b in range(30, -1, -1):
        c0 = jnp.sum(((st >> b) == (prefix >> b)).astype(jnp.int32),
                     axis=0, keepdims=True)
        cond = kk <= c0
        prefix = jnp.where(cond, prefix, prefix | (1 << b))
        kk = jnp.where(cond, kk, kk - c0)
    return prefix


def _dist_block(xb, xf):
    xxt = jax.lax.dot_general(xf, xb, (((1,), (1,)), ((), ())),
                              preferred_element_type=jnp.float32)
    xsqp = jax.lax.dot_general(jnp.ones((1, _C), jnp.float32), xb * xb,
                               (((1,), (1,)), ((), ())),
                               preferred_element_type=jnp.float32)
    xsqn = jnp.sum(xf * xf, axis=1, keepdims=True)
    D = xsqp + (-2.0 * xxt) + xsqn
    bits = jax.lax.bitcast_convert_type(D, jnp.int32)
    return jnp.where(bits < 0, bits ^ jnp.int32(0x7FFFFFFF), bits)


def _dist_deg_body(xb_ref, xf_ref, dv_ref):
    st = _dist_block(xb_ref[0], xf_ref[0])
    kvec = jnp.full((1, _BR), _K1, jnp.int32)
    v = _threshold_extract(st, _K1)
    m1 = _members_from_threshold(st, v, kvec).astype(jnp.float32)
    part = jnp.sum(m1, axis=1, keepdims=True)

    @pl.when(pl.program_id(1) == 0)
    def _init():
        dv_ref[0] = part

    @pl.when(pl.program_id(1) != 0)
    def _acc():
        dv_ref[0] += part


def _inc_gp_body(xb_ref, dvr_ref, x_ref, Wc_ref, bc_ref, L_ref, clc_ref,
                 clr_ref, P_ref, S_s, h_s, dvc_s, dvr_s):
    j = pl.program_id(1)

    @pl.when(j == 0)
    def _h():
        h = jax.lax.dot_general(x_ref[0], Wc_ref[...],
                                (((1,), (1,)), ((), ())),
                                preferred_element_type=jnp.float32)
        h_s[...] = h + bc_ref[...]

    kvec = dvr_ref[0].astype(jnp.int32)
    st = _dist_block(xb_ref[0], x_ref[0])
    v = _threshold_radix(st, kvec)
    m2 = _members_from_threshold(st, v, kvec).astype(jnp.float32)
    ri = jax.lax.broadcasted_iota(jnp.int32, (_N, _BR), 0)
    ci = jax.lax.broadcasted_iota(jnp.int32, (_N, _BR), 1) + j * _BR
    nh = jnp.where(ri == ci, 1.0, m2)
    de = jnp.sum(nh, axis=0, keepdims=True)
    aw = nh * (1.0 / de)
    spart = jax.lax.dot_general(aw, nh, (((1,), (1,)), ((), ())),
                                preferred_element_type=jnp.float32)
    dcol = jnp.sum(nh, axis=1, keepdims=True)
    drow = jax.lax.dot_general(jnp.ones((1, _BR), jnp.float32), nh,
                               (((1,), (1,)), ((), ())),
                               preferred_element_type=jnp.float32)

    @pl.when(j == 0)
    def _init():
        S_s[...] = spart
        dvc_s[...] = dcol
        dvr_s[...] = drow

    @pl.when(j != 0)
    def _acc():
        S_s[...] += spart
        dvc_s[...] += dcol
        dvr_s[...] += drow

    @pl.when(j == _NB - 1)
    def _fin():
        sc = jax.lax.rsqrt(dvc_s[...] + clc_ref[...])
        sr = jax.lax.rsqrt(dvr_s[...] + clr_ref[...])
        G = (S_s[...] + L_ref[...]) * sc * sr
        P_ref[0] = jax.lax.dot_general(G, h_s[...], (((1,), (0,)), ((), ())),
                                       preferred_element_type=jnp.float32)


def _bn_body(P_ref, x_ref, gamma_ref, beta_ref, o_ref):
    P = jnp.reshape(P_ref[...], (_B * _N, _C))
    m = jnp.mean(P, axis=0, keepdims=True)
    d = P - m
    var = jnp.mean(d * d, axis=0, keepdims=True)
    hn = d / jnp.sqrt(var + _EPS) * gamma_ref[...] + beta_ref[...]
    hr = jnp.maximum(hn, 0.0)
    o_ref[...] = jnp.reshape(hr, (_B, _N, _C)) + x_ref[...]


@jax.jit
def kernel(x, Wc, bc, gamma, beta):
    L = jnp.asarray(_LOC_L)
    clc = jnp.asarray(_LOC_CLOC).reshape(_N, 1)
    clr = jnp.asarray(_LOC_CLOC).reshape(1, _N)

    dv_col = pl.pallas_call(
        _dist_deg_body,
        grid=(_B, _NB),
        in_specs=[
            pl.BlockSpec((1, _BR, _C), lambda b, j: (b, j, 0)),
            pl.BlockSpec((1, _N, _C), lambda b, j: (b, 0, 0)),
        ],
        out_specs=pl.BlockSpec((1, _N, 1), lambda b, j: (b, 0, 0)),
        out_shape=jax.ShapeDtypeStruct((_B, _N, 1), jnp.float32),
    )(x, x)
    dv_row = jnp.swapaxes(dv_col, 1, 2)

    P = pl.pallas_call(
        _inc_gp_body,
        grid=(_B, _NB),
        in_specs=[
            pl.BlockSpec((1, _BR, _C), lambda b, j: (b, j, 0)),
            pl.BlockSpec((1, 1, _BR), lambda b, j: (b, 0, j)),
            pl.BlockSpec((1, _N, _C), lambda b, j: (b, 0, 0)),
            pl.BlockSpec((_C, _C), lambda b, j: (0, 0)),
            pl.BlockSpec((1, _C), lambda b, j: (0, 0)),
            pl.BlockSpec((_N, _N), lambda b, j: (0, 0)),
            pl.BlockSpec((_N, 1), lambda b, j: (0, 0)),
            pl.BlockSpec((1, _N), lambda b, j: (0, 0)),
        ],
        out_specs=pl.BlockSpec((1, _N, _C), lambda b, j: (b, 0, 0)),
        out_shape=jax.ShapeDtypeStruct((_B, _N, _C), jnp.float32),
        scratch_shapes=[
            pltpu.VMEM((_N, _N), jnp.float32),
            pltpu.VMEM((_N, _C), jnp.float32),
            pltpu.VMEM((_N, 1), jnp.float32),
            pltpu.VMEM((1, _N), jnp.float32),
        ],
    )(x, dv_row, x, Wc, bc.reshape(1, _C), L, clc, clr)

    out = pl.pallas_call(
        _bn_body,
        out_shape=jax.ShapeDtypeStruct((_B, _N, _C), jnp.float32),
    )(P, x, gamma.reshape(1, _C), beta.reshape(1, _C))
    return out


# speedup vs baseline: 1.5705x; 1.0133x over previous
import numpy as np
import jax
import jax.numpy as jnp
from jax.experimental import pallas as pl
from jax.experimental.pallas import tpu as pltpu

_B, _NODE, _C = 4, 32, 256
_N = _NODE * _NODE
_K1 = 11
_KS, _STRIDE = 5, 2
_EPS = 1e-5
_BR = 256
_NB = _N // _BR


def _local_parts():
    size, ks, stride = _NODE, _KS, _STRIDE
    inp = np.arange(size * size).reshape(size, size)
    patches = []
    for i in range(0, size - ks + 1, stride):
        for j in range(0, size - ks + 1, stride):
            patches.append(inp[i:i + ks, j:j + ks].reshape(-1))
    inp_unf = np.stack(patches, axis=0)
    edge = inp_unf.shape[0]
    H = np.zeros((size * size, edge), dtype=np.float32)
    for e in range(edge):
        H[inp_unf[e], e] = 1.0
    L = (H @ H.T) / float(ks * ks)
    cloc = H.sum(axis=1)
    return L.astype(np.float32), cloc.astype(np.float32)


_LOC_L, _LOC_CLOC = _local_parts()


def _members_from_threshold(st, v, kvec, c_lt):
    n, p = st.shape
    lt = st < v
    eqm = (st == v).astype(jnp.int32)
    c_eq = jnp.sum(eqm, axis=0, keepdims=True)
    simple = jnp.all(c_lt + c_eq == kvec)

    def _zeros():
        return jnp.zeros((n, p), jnp.int32)

    def _scan():
        csum = eqm
        d = 1
        while d < n:
            shifted = jnp.concatenate(
                [jnp.zeros((d, p), jnp.int32), csum[: n - d, :]], axis=0)
            csum = csum + shifted
            d *= 2
        return csum - eqm

    excl = jax.lax.cond(simple, _zeros, _scan)
    tie_take = eqm.astype(jnp.bool_) & (excl < (kvec - c_lt))
    return lt | tie_take


def _threshold_extract(st, k):
    rem = st
    removed = jnp.zeros((1, st.shape[1]), jnp.int32)
    v = jnp.full((1, st.shape[1]), -2147483648, jnp.int32)
    c_lt = jnp.zeros((1, st.shape[1]), jnp.int32)
    for _ in range(k):
        cur = jnp.min(rem, axis=0, keepdims=True)
        isv = rem == cur
        ccur = jnp.sum(isv.astype(jnp.int32), axis=0, keepdims=True)
        upd = removed < k
        v = jnp.where(upd, cur, v)
        c_lt = jnp.where(upd, removed, c_lt)
        removed = removed + jnp.where(upd, ccur, 0)
        rem = jnp.where(isv, jnp.int32(2147483647), rem)
    return v, c_lt


def _threshold_radix(st, kvec):
    neg = (st < 0).astype(jnp.int32)
    c0 = jnp.sum(neg, axis=0, keepdims=True)
    cond = kvec <= c0
    prefix = jnp.where(cond, jnp.int32(-2147483648), jnp.int32(0))
    kk = jnp.where(cond, kvec, kvec - c0)
    for b in range(30, -1, -1):
        c0 = jnp.sum(((st >> b) == (prefix >> b)).astype(jnp.int32),
                     axis=0, keepdims=True)
        cond = kk <= c0
        prefix = jnp.where(cond, prefix, prefix | (1 << b))
        kk = jnp.where(cond, kk, kk - c0)
    return prefix, kvec - kk


def _dist_block(xb, xf):
    xxt = jax.lax.dot_general(xf, xb, (((1,), (1,)), ((), ())),
                              preferred_element_type=jnp.float32)
    xsqp = jax.lax.dot_general(jnp.ones((1, _C), jnp.float32), xb * xb,
                               (((1,), (1,)), ((), ())),
                               preferred_element_type=jnp.float32)
    xsqn = jnp.sum(xf * xf, axis=1, keepdims=True)
    D = xsqp + (-2.0 * xxt) + xsqn
    bits = jax.lax.bitcast_convert_type(D, jnp.int32)
    return jnp.where(bits < 0, bits ^ jnp.int32(0x7FFFFFFF), bits)


def _dist_deg_body(xb_ref, xf_ref, dv_ref):
    st = _dist_block(xb_ref[0], xf_ref[0])
    kvec = jnp.full((1, _BR), _K1, jnp.int32)
    v, c_lt = _threshold_extract(st, _K1)
    m1 = _members_from_threshold(st, v, kvec, c_lt).astype(jnp.float32)
    part = jnp.sum(m1, axis=1, keepdims=True)

    @pl.when(pl.program_id(1) == 0)
    def _init():
        dv_ref[0] = part

    @pl.when(pl.program_id(1) != 0)
    def _acc():
        dv_ref[0] += part


def _inc_gp_body(xb_ref, dvr_ref, x_ref, Wc_ref, bc_ref, L_ref, clc_ref,
                 clr_ref, P_ref, S_s, h_s, dvc_s, dvr_s):
    j = pl.program_id(1)

    @pl.when(j == 0)
    def _h():
        h = jax.lax.dot_general(x_ref[0], Wc_ref[...],
                                (((1,), (1,)), ((), ())),
                                preferred_element_type=jnp.float32)
        h_s[...] = h + bc_ref[...]

    kvec = dvr_ref[0].astype(jnp.int32)
    st = _dist_block(xb_ref[0], x_ref[0])
    v, c_lt = _threshold_radix(st, kvec)
    m2 = _members_from_threshold(st, v, kvec, c_lt).astype(jnp.float32)
    ri = jax.lax.broadcasted_iota(jnp.int32, (_N, _BR), 0)
    ci = jax.lax.broadcasted_iota(jnp.int32, (_N, _BR), 1) + j * _BR
    nh = jnp.where(ri == ci, 1.0, m2)
    de = jnp.sum(nh, axis=0, keepdims=True)
    aw = nh * (1.0 / de)
    spart = jax.lax.dot_general(aw, nh, (((1,), (1,)), ((), ())),
                                preferred_element_type=jnp.float32)
    dcol = jnp.sum(nh, axis=1, keepdims=True)
    drow = jax.lax.dot_general(jnp.ones((1, _BR), jnp.float32), nh,
                               (((1,), (1,)), ((), ())),
                               preferred_element_type=jnp.float32)

    @pl.when(j == 0)
    def _init():
        S_s[...] = spart
        dvc_s[...] = dcol
        dvr_s[...] = drow

    @pl.when(j != 0)
    def _acc():
        S_s[...] += spart
        dvc_s[...] += dcol
        dvr_s[...] += drow

    @pl.when(j == _NB - 1)
    def _fin():
        sc = jax.lax.rsqrt(dvc_s[...] + clc_ref[...])
        sr = jax.lax.rsqrt(dvr_s[...] + clr_ref[...])
        G = (S_s[...] + L_ref[...]) * sc * sr
        P_ref[0] = jax.lax.dot_general(G, h_s[...], (((1,), (0,)), ((), ())),
                                       preferred_element_type=jnp.float32)


def _bn_body(P_ref, x_ref, gamma_ref, beta_ref, o_ref):
    P = jnp.reshape(P_ref[...], (_B * _N, _C))
    m = jnp.mean(P, axis=0, keepdims=True)
    d = P - m
    var = jnp.mean(d * d, axis=0, keepdims=True)
    hn = d / jnp.sqrt(var + _EPS) * gamma_ref[...] + beta_ref[...]
    hr = jnp.maximum(hn, 0.0)
    o_ref[...] = jnp.reshape(hr, (_B, _N, _C)) + x_ref[...]


@jax.jit
def kernel(x, Wc, bc, gamma, beta):
    L = jnp.asarray(_LOC_L)
    clc = jnp.asarray(_LOC_CLOC).reshape(_N, 1)
    clr = jnp.asarray(_LOC_CLOC).reshape(1, _N)

    dv_col = pl.pallas_call(
        _dist_deg_body,
        grid=(_B, _NB),
        in_specs=[
            pl.BlockSpec((1, _BR, _C), lambda b, j: (b, j, 0)),
            pl.BlockSpec((1, _N, _C), lambda b, j: (b, 0, 0)),
        ],
        out_specs=pl.BlockSpec((1, _N, 1), lambda b, j: (b, 0, 0)),
        out_shape=jax.ShapeDtypeStruct((_B, _N, 1), jnp.float32),
    )(x, x)
    dv_row = jnp.swapaxes(dv_col, 1, 2)

    P = pl.pallas_call(
        _inc_gp_body,
        grid=(_B, _NB),
        in_specs=[
            pl.BlockSpec((1, _BR, _C), lambda b, j: (b, j, 0)),
            pl.BlockSpec((1, 1, _BR), lambda b, j: (b, 0, j)),
            pl.BlockSpec((1, _N, _C), lambda b, j: (b, 0, 0)),
            pl.BlockSpec((_C, _C), lambda b, j: (0, 0)),
            pl.BlockSpec((1, _C), lambda b, j: (0, 0)),
            pl.BlockSpec((_N, _N), lambda b, j: (0, 0)),
            pl.BlockSpec((_N, 1), lambda b, j: (0, 0)),
            pl.BlockSpec((1, _N), lambda b, j: (0, 0)),
        ],
        out_specs=pl.BlockSpec((1, _N, _C), lambda b, j: (b, 0, 0)),
        out_shape=jax.ShapeDtypeStruct((_B, _N, _C), jnp.float32),
        scratch_shapes=[
            pltpu.VMEM((_N, _N), jnp.float32),
            pltpu.VMEM((_N, _C), jnp.float32),
            pltpu.VMEM((_N, 1), jnp.float32),
            pltpu.VMEM((1, _N), jnp.float32),
        ],
    )(x, dv_row, x, Wc, bc.reshape(1, _C), L, clc, clr)

    out = pl.pallas_call(
        _bn_body,
        out_shape=jax.ShapeDtypeStruct((_B, _N, _C), jnp.float32),
    )(P, x, gamma.reshape(1, _C), beta.reshape(1, _C))
    return out
